# Initial kernel scaffold; baseline (speedup 1.0000x reference)
import jax
import jax.numpy as jnp
from jax.experimental import pallas as pl


def kernel(u, i, user_df, item_df, table):
    raise NotImplementedError("write your pallas kernel here")


# speedup vs baseline: 1.6518x; 1.6518x over previous
import functools

import jax
import jax.numpy as jnp
from jax import lax
from jax.experimental import pallas as pl
from jax.experimental.pallas import tpu as pltpu
from jax.experimental.pallas import tpu_sc as plsc

B = 16384
DIM = 32
N_USERS = 100000
F = 13
NF = 2 * F

NC, NS, L = 2, 16, 16
NW = NC * NS
SPT = B // NW
GROUP = 32
NGROUPS = SPT // GROUP
RPG = GROUP * NF


def _rsqrt(n2):
    bits = lax.bitcast_convert_type(n2, jnp.int32)
    y = lax.bitcast_convert_type(jnp.int32(0x5F3759DF) - (bits >> 1), jnp.float32)
    h = 0.5 * n2
    for _ in range(3):
        y = y * (1.5 - h * y * y)
    return y


@functools.partial(
    pl.kernel,
    out_type=jax.ShapeDtypeStruct((B,), jnp.float32),
    mesh=plsc.VectorSubcoreMesh(core_axis_name="c", subcore_axis_name="s"),
    compiler_params=pltpu.CompilerParams(
        needs_layout_passes=False, use_tc_tiling_on_sc=False
    ),
    scratch_types=[
        pltpu.VMEM((SPT * 2,), jnp.int32),
        pltpu.VMEM((SPT * 2, 16), jnp.int32),
        pltpu.VMEM((RPG + 8,), jnp.int32),
        pltpu.VMEM((RPG + 8,), jnp.int32),
        pltpu.VMEM((2, RPG, DIM), jnp.float32),
        pltpu.VMEM((DIM, L), jnp.float32),
        pltpu.VMEM((SPT,), jnp.float32),
        pltpu.SemaphoreType.DMA,
        pltpu.SemaphoreType.DMA,
    ],
)
def _fm_sc(pair_hbm, ftab_hbm, table_hbm, out_hbm, pairv, featv, idx0, idx1, emb, sacc, outv, sem0, sem1):
    wid = lax.axis_index("s") * NC + lax.axis_index("c")
    sems = (sem0, sem1)
    idxs = (idx0, idx1)

    pltpu.sync_copy(pair_hbm.at[pl.ds(wid * (SPT * 2), SPT * 2)], pairv)
    pltpu.async_copy(ftab_hbm.at[pairv], featv, sem0).wait()

    def repack(g, b):
        ib = idxs[b]
        for s in range(GROUP):
            r = g * (GROUP * 2) + 2 * s
            uv = featv[r, :]
            iv = featv[r + 1, :]
            ib[pl.ds(s * NF, L)] = uv
            ib[pl.ds(s * NF + F, L)] = iv

    def g2_start(g, b):
        del g
        pltpu.async_copy(
            table_hbm.at[idxs[b].at[pl.ds(0, RPG)]], emb.at[b], sems[b]
        )

    def g2_wait(b):
        pltpu.make_async_copy(
            table_hbm.at[idxs[b].at[pl.ds(0, RPG)]], emb.at[b], sems[b]
        ).wait()

    iota26 = lax.iota(jnp.int32, L) * NF
    zero = jnp.zeros((L,), jnp.float32)

    def chunk_compute(b, g, c):
        embref = emb.at[b]
        for d in range(DIM):
            sacc[d, :] = zero

        def f_body(f, ssq):
            rowv = iota26 + (c * (L * NF) + f)
            vs = []
            n2 = zero
            for d in range(DIM):
                colv = jnp.full((L,), d, jnp.int32)
                v = plsc.load_gather(embref, [rowv, colv])
                vs.append(v)
                n2 = n2 + v * v
            r = _rsqrt(n2)
            scale = jnp.where(n2 > 1.0, r, 1.0)
            for d in range(DIM):
                plsc.addupdate(sacc.at[d], scale * vs[d])
            return ssq + scale * scale * n2

        ssq = lax.fori_loop(0, NF, f_body, zero)
        acc = zero
        for d in range(DIM):
            sd = sacc[d, :]
            acc = acc + sd * sd
        out = 0.5 * (acc - ssq)
        logit = 1.0 / (1.0 + jnp.exp(-out))
        outv[pl.ds(g * GROUP + c * L, L)] = logit

    repack(0, 0)
    g2_start(0, 0)
    repack(1, 1)
    g2_start(1, 1)

    def two_groups(t, carry):
        for b in (0, 1):
            g = t * 2 + b
            g2_wait(b)
            chunk_compute(b, g, 0)
            chunk_compute(b, g, 1)

            @pl.when(g + 2 < NGROUPS)
            def _():
                repack(g + 2, b)
                g2_start(g + 2, b)

        return carry

    lax.fori_loop(0, NGROUPS // 2, two_groups, 0)
    pltpu.sync_copy(outv, out_hbm.at[pl.ds(wid * SPT, SPT)])


def kernel(u, i, user_df, item_df, table):
    u = u.astype(jnp.int32)
    i = i.astype(jnp.int32)
    ftab = jnp.concatenate(
        [user_df.astype(jnp.int32), item_df.astype(jnp.int32)], axis=0
    )
    ftab = jnp.pad(ftab, ((0, 0), (0, 16 - F)))
    pair = jnp.stack([u, i + N_USERS], axis=1).reshape(B * 2)
    return _fm_sc(pair, ftab, table)
